# baseline (speedup 1.0000x reference)
import functools

import jax
import jax.numpy as jnp
from jax import lax
from jax.experimental import pallas as pl
from jax.experimental.pallas import tpu as pltpu
from jax.experimental.pallas import tpu_sc as plsc

_SEQ = 200
_D = 128
_NCHUNK = _D // 16


def _sc_body(x0_hbm, x1_hbm, emb1_hbm, emb2_hbm, w1_hbm, b1_hbm, out_hbm,
             idx16_v, rows16_v, idx8_v, rows8_v, idx2_v, row2_v,
             acc_v, local_v, w1_v, b1_v, cv_v, cl_v, outv_v,
             stage_s, cstage_s,
             sem_g, sem_w1):
    w = lax.axis_index("s")
    c = lax.axis_index("c")

    w1_dma = pltpu.make_async_copy(w1_hbm.at[pl.ds(w * 8, 8)], w1_v, sem_w1)
    w1_dma.start()

    @pl.when(w < 9)
    def _gather16():
        base = w * 16
        pltpu.sync_copy(x0_hbm.at[pl.ds(base, 16)], idx16_v)
        pltpu.async_copy(emb1_hbm.at[idx16_v], rows16_v, sem_g).wait()
        for ci in range(_NCHUNK):
            acc = rows16_v[0, pl.ds(ci * 16, 16)]
            for r in range(1, 16):
                acc = acc + rows16_v[r, pl.ds(ci * 16, 16)]
            acc_v[pl.ds(ci * 16, 16)] = acc
        pltpu.sync_copy(acc_v, stage_s.at[w])

    @pl.when(w >= 9)
    def _gather8():
        base = w * 8 + 72
        pltpu.sync_copy(x0_hbm.at[pl.ds(base, 8)], idx8_v)
        pltpu.async_copy(emb1_hbm.at[idx8_v], rows8_v, sem_g).wait()
        for ci in range(_NCHUNK):
            acc = rows8_v[0, pl.ds(ci * 16, 16)]
            for r in range(1, 8):
                acc = acc + rows8_v[r, pl.ds(ci * 16, 16)]
            acc_v[pl.ds(ci * 16, 16)] = acc
        pltpu.sync_copy(acc_v, stage_s.at[w])

    @pl.when(w == 15)
    def _gather_emb2():
        pltpu.sync_copy(x1_hbm, idx2_v)
        pltpu.async_copy(emb2_hbm.at[idx2_v], row2_v, sem_g).wait()
        pltpu.sync_copy(row2_v, stage_s.at[pl.ds(16, 1)])

    plsc.subcore_barrier()

    pltpu.sync_copy(stage_s, local_v)
    inv = 1.0 / _SEQ
    m = []
    v = []
    for ci in range(_NCHUNK):
        s = local_v[0, pl.ds(ci * 16, 16)]
        for p in range(1, 16):
            s = s + local_v[p, pl.ds(ci * 16, 16)]
        m.append(s * inv)
        v.append(local_v[16, pl.ds(ci * 16, 16)])

    w1_dma.wait()

    c0 = w // 2
    off = (w % 2) * 8
    vchunk = jnp.zeros((16,), jnp.float32)
    for ci in range(_NCHUNK):
        vchunk = jnp.where(ci == c0, v[ci], vchunk)

    t = [jnp.zeros((16,), jnp.float32) for _ in range(_NCHUNK)]
    for r in range(8):
        bidx = jnp.full((16,), off + r, jnp.int32)
        bv = vchunk.at[bidx].get(mode="promise_in_bounds")
        for ci in range(_NCHUNK):
            t[ci] = t[ci] + bv * w1_v[r, pl.ds(ci * 16, 16)]
    pvec = t[0] * m[0]
    for ci in range(1, _NCHUNK):
        pvec = pvec + t[ci] * m[ci]

    @pl.when(w == 15)
    def _bias_term():
        pltpu.sync_copy(b1_hbm, b1_v)
        bias = b1_v[pl.ds(0, 16)] * v[0]
        for ci in range(1, _NCHUNK):
            bias = bias + b1_v[pl.ds(ci * 16, 16)] * v[ci]
        cv_v[...] = pvec + bias

    @pl.when(w < 15)
    def _no_bias():
        cv_v[...] = pvec

    pltpu.sync_copy(cv_v, cstage_s.at[w])
    plsc.subcore_barrier()

    @pl.when(jnp.logical_and(w == 0, c == 0))
    def _finish():
        pltpu.sync_copy(cstage_s, cl_v)
        tot = cl_v[0, pl.ds(0, 16)]
        for p in range(1, 16):
            tot = tot + cl_v[p, pl.ds(0, 16)]
        lane = lax.iota(jnp.int32, 16)
        for step in (1, 2, 4, 8):
            ridx = (lane + step) & 15
            tot = tot + tot.at[ridx].get(mode="promise_in_bounds")
        y = 1.0 / (1.0 + jnp.exp(-tot))
        outv_v[...] = y
        pltpu.sync_copy(outv_v, out_hbm)


_sc_all = functools.partial(
    pl.kernel,
    _sc_body,
    out_type=jax.ShapeDtypeStruct((16,), jnp.float32),
    scratch_types=[
        pltpu.VMEM((16,), jnp.int32),
        pltpu.VMEM((16, _D), jnp.float32),
        pltpu.VMEM((8,), jnp.int32),
        pltpu.VMEM((8, _D), jnp.float32),
        pltpu.VMEM((1,), jnp.int32),
        pltpu.VMEM((1, _D), jnp.float32),
        pltpu.VMEM((_D,), jnp.float32),
        pltpu.VMEM((17, _D), jnp.float32),
        pltpu.VMEM((8, _D), jnp.float32),
        pltpu.VMEM((_D,), jnp.float32),
        pltpu.VMEM((16,), jnp.float32),
        pltpu.VMEM((16, 16), jnp.float32),
        pltpu.VMEM((16,), jnp.float32),
        pltpu.VMEM_SHARED((17, _D), jnp.float32),
        pltpu.VMEM_SHARED((16, 16), jnp.float32),
        pltpu.SemaphoreType.DMA,
        pltpu.SemaphoreType.DMA,
    ],
    mesh=plsc.VectorSubcoreMesh(core_axis_name="c", subcore_axis_name="s"),
)()


def kernel(x0, x1, emb1, W1, b1, emb2):
    out = _sc_all(x0, x1, emb1, emb2, W1, b1)
    return out[0]


# speedup vs baseline: 1.1209x; 1.1209x over previous
import functools

import jax
import jax.numpy as jnp
from jax import lax
from jax.experimental import pallas as pl
from jax.experimental.pallas import tpu as pltpu
from jax.experimental.pallas import tpu_sc as plsc

_SEQ = 200
_D = 128
_NCHUNK = _D // 16
_NPART = 13


def _row_sum(rows_v, nrows, acc_v):
    def body(r, carry):
        return tuple(carry[ci] + rows_v[r, pl.ds(ci * 16, 16)]
                     for ci in range(_NCHUNK))
    init = tuple(rows_v[0, pl.ds(ci * 16, 16)] for ci in range(_NCHUNK))
    acc = lax.fori_loop(1, nrows, body, init)
    for ci in range(_NCHUNK):
        acc_v[pl.ds(ci * 16, 16)] = acc[ci]


def _sc_body(x0_hbm, x1_hbm, emb1_hbm, emb2_hbm, out_hbm,
             idx_v, rows_v, acc_v, idx2_v, row2_v, sem):
    w = lax.axis_index("s")

    @pl.when(w < 12)
    def _gather16():
        pltpu.sync_copy(x0_hbm.at[pl.ds(w * 16, 16)], idx_v)
        pltpu.async_copy(emb1_hbm.at[idx_v], rows_v, sem).wait()
        _row_sum(rows_v, 16, acc_v)
        pltpu.sync_copy(acc_v, out_hbm.at[w])

    @pl.when(w == 12)
    def _gather8():
        pltpu.sync_copy(x0_hbm.at[pl.ds(192, 8)], idx_v.at[pl.ds(0, 8)])
        pltpu.async_copy(emb1_hbm.at[idx_v.at[pl.ds(0, 8)]],
                         rows_v.at[pl.ds(0, 8)], sem).wait()
        _row_sum(rows_v, 8, acc_v)
        pltpu.sync_copy(acc_v, out_hbm.at[w])

    @pl.when(w == 13)
    def _gather_emb2():
        pltpu.sync_copy(x1_hbm, idx2_v)
        pltpu.async_copy(emb2_hbm.at[idx2_v], row2_v, sem).wait()
        pltpu.sync_copy(row2_v, out_hbm.at[pl.ds(_NPART, 1)])


_sc_gather = functools.partial(
    pl.kernel,
    _sc_body,
    out_type=jax.ShapeDtypeStruct((_NPART + 1, _D), jnp.float32),
    scratch_types=[
        pltpu.VMEM((16,), jnp.int32),
        pltpu.VMEM((16, _D), jnp.float32),
        pltpu.VMEM((_D,), jnp.float32),
        pltpu.VMEM((1,), jnp.int32),
        pltpu.VMEM((1, _D), jnp.float32),
        pltpu.SemaphoreType.DMA,
    ],
    mesh=plsc.VectorSubcoreMesh(core_axis_name="c", subcore_axis_name="s",
                                num_cores=1),
)()


def _tc_dense_body(stage_ref, w1_ref, b1_ref, out_ref):
    parts = stage_ref[0:_NPART, :]
    m = jnp.sum(parts, axis=0, keepdims=True) * (1.0 / _SEQ)
    y1 = lax.dot_general(m, w1_ref[...], (((1,), (1,)), ((), ())),
                         preferred_element_type=jnp.float32)
    y1 = y1 + b1_ref[...]
    v = stage_ref[_NPART:_NPART + 1, :]
    s = jnp.sum(y1 * v)
    out_ref[...] = jax.nn.sigmoid(s) * jnp.ones((1, _D), jnp.float32)


def kernel(x0, x1, emb1, W1, b1, emb2):
    stage = _sc_gather(x0, x1, emb1, emb2)
    out = pl.pallas_call(
        _tc_dense_body,
        out_shape=jax.ShapeDtypeStruct((1, _D), jnp.float32),
    )(stage, W1, b1.reshape(1, _D))
    return out[0, 0]


# speedup vs baseline: 1.1346x; 1.0122x over previous
import functools

import jax
import jax.numpy as jnp
from jax import lax
from jax.experimental import pallas as pl
from jax.experimental.pallas import tpu as pltpu
from jax.experimental.pallas import tpu_sc as plsc

_SEQ = 200
_D = 128
_NCHUNK = _D // 16
_NPART = 13


def _sc_body(x0_hbm, emb1_hbm, out_hbm, idx_v, rows_v, acc_v, sem):
    w = lax.axis_index("s")

    @pl.when(w < _NPART)
    def _gather():
        base = jnp.minimum(w * 16, _SEQ - 16)
        lo = w * 16 - base
        pltpu.sync_copy(x0_hbm.at[pl.ds(base, 16)], idx_v)
        pltpu.async_copy(emb1_hbm.at[idx_v], rows_v, sem).wait()

        def body(r, carry):
            wt = jnp.where(r >= lo, 1.0, 0.0)
            return tuple(carry[ci] + wt * rows_v[r, pl.ds(ci * 16, 16)]
                         for ci in range(_NCHUNK))

        zero = jnp.zeros((16,), jnp.float32)
        acc = lax.fori_loop(0, 16, body, (zero,) * _NCHUNK)
        for ci in range(_NCHUNK):
            acc_v[pl.ds(ci * 16, 16)] = acc[ci]
        pltpu.sync_copy(acc_v, out_hbm.at[w])


_sc_gather = functools.partial(
    pl.kernel,
    _sc_body,
    out_type=jax.ShapeDtypeStruct((_NPART, _D), jnp.float32),
    scratch_types=[
        pltpu.VMEM((16,), jnp.int32),
        pltpu.VMEM((16, _D), jnp.float32),
        pltpu.VMEM((_D,), jnp.float32),
        pltpu.SemaphoreType.DMA,
    ],
    mesh=plsc.VectorSubcoreMesh(core_axis_name="c", subcore_axis_name="s",
                                num_cores=1),
)()


def _tc_dense_body(x1_ref, stage_ref, w1_ref, b1_ref, v_ref, out_ref):
    m = jnp.sum(stage_ref[...], axis=0, keepdims=True) * (1.0 / _SEQ)
    y1 = lax.dot_general(m, w1_ref[...], (((1,), (1,)), ((), ())),
                         preferred_element_type=jnp.float32)
    y1 = y1 + b1_ref[...]
    s = jnp.sum(y1 * v_ref[0])
    out_ref[...] = jax.nn.sigmoid(s) * jnp.ones((1, _D), jnp.float32)


def kernel(x0, x1, emb1, W1, b1, emb2):
    stage = _sc_gather(x0, emb1)
    grid_spec = pltpu.PrefetchScalarGridSpec(
        num_scalar_prefetch=1,
        grid=(1,),
        in_specs=[
            pl.BlockSpec((_NPART, _D), lambda i, x1r: (0, 0)),
            pl.BlockSpec((_D, _D), lambda i, x1r: (0, 0)),
            pl.BlockSpec((1, _D), lambda i, x1r: (0, 0)),
            pl.BlockSpec((1, 1, _D), lambda i, x1r: (x1r[0], 0, 0)),
        ],
        out_specs=pl.BlockSpec((1, _D), lambda i, x1r: (0, 0)),
    )
    out = pl.pallas_call(
        _tc_dense_body,
        grid_spec=grid_spec,
        out_shape=jax.ShapeDtypeStruct((1, _D), jnp.float32),
    )(x1, stage, W1, b1.reshape(1, _D), emb2.reshape(-1, 1, _D))
    return out[0, 0]
